# baseline (speedup 1.0000x reference)
import functools

import jax
import jax.numpy as jnp
from jax.experimental import pallas as pl
from jax.experimental.pallas import tpu as pltpu
from jax.experimental.pallas import tpu_sc as plsc


def _conv2d(x, w, b, stride, pad):
    out = jax.lax.conv_general_dilated(
        x, w, (stride, stride), [(pad, pad), (pad, pad)],
        dimension_numbers=('NCHW', 'OIHW', 'NCHW'))
    if b is not None:
        out = out + b[None, :, None, None]
    return out


def _conv_transpose2d(x, w, b, stride, pad):
    w_t = jnp.transpose(w[:, :, ::-1, ::-1], (1, 0, 2, 3))
    k = w.shape[2]
    p = k - 1 - pad
    out = jax.lax.conv_general_dilated(
        x, w_t, (1, 1), [(p, p), (p, p)], lhs_dilation=(stride, stride),
        dimension_numbers=('NCHW', 'OIHW', 'NCHW'))
    if b is not None:
        out = out + b[None, :, None, None]
    return out


def _residual(x, wa, wb):
    h = jax.nn.relu(_conv2d(x, wa, None, 1, 1))
    return jax.nn.relu(_conv2d(h, wb, None, 1, 0))


_R = 512


def _vq_block_kernel(flat_ref, emb_ref, idx_ref, minsum_ref, counts_ref, *,
                     n_valid):
    i = pl.program_id(0)

    @pl.when(i == 0)
    def _init():
        minsum_ref[...] = jnp.zeros_like(minsum_ref)
        counts_ref[...] = jnp.zeros_like(counts_ref)

    f = flat_ref[...]
    e = emb_ref[...]
    rn = jnp.sum(f * f, axis=1, keepdims=True)
    en = jnp.sum(e * e, axis=1)
    mm = jax.lax.dot_general(f, e, (((1,), (1,)), ((), ())),
                             preferred_element_type=jnp.float32)
    d = rn + en[None, :] - 2.0 * mm
    minval = jnp.min(d, axis=1, keepdims=True)
    cidx = jax.lax.broadcasted_iota(jnp.int32, d.shape, 1)
    idx = jnp.min(jnp.where(d == minval, cidx, d.shape[1]), axis=1,
                  keepdims=True)
    idx_ref[...] = idx
    onehot = (cidx == idx).astype(jnp.float32)
    rows = i * _R + jax.lax.broadcasted_iota(jnp.int32, (_R, 1), 0)
    vmask = (rows < n_valid).astype(jnp.float32)
    minsum_ref[...] += jnp.sum(minval * vmask).reshape(1, 1)
    counts_ref[...] += jnp.sum(onehot * vmask, axis=0)[None, :]


@functools.cache
def _sc_gather(b_total, d_model):
    info = plsc.get_sparse_core_info()
    nw = info.num_cores * info.num_subcores
    b_per_w = b_total // nw
    chunk = min(b_per_w, 128)
    n_chunks = b_per_w // chunk
    mesh = plsc.VectorSubcoreMesh(core_axis_name="c", subcore_axis_name="s")

    @functools.partial(
        pl.kernel, mesh=mesh,
        out_type=jax.ShapeDtypeStruct((b_total, d_model), jnp.float32),
        scratch_types=[
            pltpu.VMEM((b_per_w,), jnp.int32),
            pltpu.VMEM((b_per_w, d_model), jnp.float32),
            pltpu.SemaphoreType.DMA,
        ],
    )
    def gather(table_hbm, idx_hbm, out_hbm, idx_v, rows_v, sem):
        wid = jax.lax.axis_index("s") * info.num_cores + jax.lax.axis_index("c")
        base = wid * b_per_w
        pltpu.sync_copy(idx_hbm.at[pl.ds(base, b_per_w)], idx_v)
        copies = [
            pltpu.async_copy(
                table_hbm.at[idx_v.at[pl.ds(c * chunk, chunk)]],
                rows_v.at[pl.ds(c * chunk, chunk), :], sem)
            for c in range(n_chunks)
        ]
        for cp in copies:
            cp.wait()
        pltpu.sync_copy(rows_v, out_hbm.at[pl.ds(base, b_per_w)])

    return gather


def _vq_quantize(flat, emb):
    n, dim = flat.shape
    k = emb.shape[0]
    npad = (-n) % _R
    flat_p = jnp.pad(flat, ((0, npad), (0, 0)))
    n_p = n + npad
    idx2, minsum, counts = pl.pallas_call(
        functools.partial(_vq_block_kernel, n_valid=n),
        grid=(n_p // _R,),
        in_specs=[pl.BlockSpec((_R, dim), lambda i: (i, 0)),
                  pl.BlockSpec((k, dim), lambda i: (0, 0))],
        out_specs=[pl.BlockSpec((_R, 1), lambda i: (i, 0)),
                   pl.BlockSpec((1, 1), lambda i: (0, 0)),
                   pl.BlockSpec((1, k), lambda i: (0, 0))],
        out_shape=[jax.ShapeDtypeStruct((n_p, 1), jnp.int32),
                   jax.ShapeDtypeStruct((1, 1), jnp.float32),
                   jax.ShapeDtypeStruct((1, k), jnp.float32)],
    )(flat_p, emb)
    emb_q = emb.astype(jnp.bfloat16).astype(jnp.float32)
    emb_wide = jnp.pad(emb_q, ((0, 0), (0, 128 - dim)))
    q = _sc_gather(n_p, 128)(emb_wide, idx2.reshape(n_p))
    return q[:n, :dim], minsum[0, 0], counts[0]


def _vq(z, emb):
    x = jnp.transpose(z, (0, 2, 3, 1))
    shp = x.shape
    flat = x.reshape(-1, emb.shape[1])
    n = flat.shape[0]
    q, minsum, counts = _vq_quantize(flat, emb)
    loss = 1.25 * (minsum / (n * emb.shape[1]))
    qr = q.reshape(shp)
    q_st = x + (qr - x)
    avg = counts / n
    perp = jnp.exp(-jnp.sum(avg * jnp.log(avg + 1e-10)))
    return loss, jnp.transpose(q_st, (0, 3, 1, 2)), perp


def kernel(x, params):
    p = params
    h = jax.nn.relu(_conv2d(x, p['eb_c1_w'], p['eb_c1_b'], 2, 1))
    h = jax.nn.relu(_conv2d(h, p['eb_c2_w'], p['eb_c2_b'], 2, 1))
    h = jax.nn.relu(_conv2d(h, p['eb_c3_w'], p['eb_c3_b'], 1, 1))
    h = _residual(h, p['eb_r1a_w'], p['eb_r1b_w'])
    z_bottom = _residual(h, p['eb_r2a_w'], p['eb_r2b_w'])
    h = jax.nn.relu(_conv2d(z_bottom, p['et_c1_w'], p['et_c1_b'], 2, 1))
    h = jax.nn.relu(_conv2d(h, p['et_c2_w'], p['et_c2_b'], 1, 1))
    h = _residual(h, p['et_r1a_w'], p['et_r1b_w'])
    z_top = _residual(h, p['et_r2a_w'], p['et_r2b_w'])
    loss_top, q_top, pt = _vq(_conv2d(z_top, p['pvt_w'], p['pvt_b'], 1, 0),
                              p['emb_top'])
    h = _conv2d(q_top, p['dt_c1_w'], p['dt_c1_b'], 1, 1)
    h = _residual(h, p['dt_r1a_w'], p['dt_r1b_w'])
    h = _residual(h, p['dt_r2a_w'], p['dt_r2b_w'])
    rec_top = _conv_transpose2d(h, p['dt_t1_w'], p['dt_t1_b'], 2, 1)
    zb = jnp.concatenate([rec_top, z_bottom], axis=1)
    loss_bottom, q_bot, pb = _vq(_conv2d(zb, p['pvb_w'], p['pvb_b'], 1, 0),
                                 p['emb_bot'])
    up = _conv_transpose2d(q_top, p['up_w'], p['up_b'], 2, 1)
    quantized = jnp.concatenate([up, q_bot], axis=1)
    h = _conv2d(quantized, p['db_c1_w'], p['db_c1_b'], 1, 1)
    h = _residual(h, p['db_r1a_w'], p['db_r1b_w'])
    h = _residual(h, p['db_r2a_w'], p['db_r2b_w'])
    h = jax.nn.relu(_conv_transpose2d(h, p['db_t1_w'], p['db_t1_b'], 2, 1))
    x_rec = _conv_transpose2d(h, p['db_t2_w'], p['db_t2_b'], 2, 1)
    return loss_top + loss_bottom, x_rec, pt + pb, quantized


# speedup vs baseline: 1.5653x; 1.5653x over previous
import functools

import jax
import jax.numpy as jnp
from jax.experimental import pallas as pl
from jax.experimental.pallas import tpu as pltpu
from jax.experimental.pallas import tpu_sc as plsc


def _conv2d(x, w, b, stride, pad):
    out = jax.lax.conv_general_dilated(
        x, w, (stride, stride), [(pad, pad), (pad, pad)],
        dimension_numbers=('NCHW', 'OIHW', 'NCHW'))
    if b is not None:
        out = out + b[None, :, None, None]
    return out


def _conv_transpose2d(x, w, b, stride, pad):
    w_t = jnp.transpose(w[:, :, ::-1, ::-1], (1, 0, 2, 3))
    k = w.shape[2]
    p = k - 1 - pad
    out = jax.lax.conv_general_dilated(
        x, w_t, (1, 1), [(p, p), (p, p)], lhs_dilation=(stride, stride),
        dimension_numbers=('NCHW', 'OIHW', 'NCHW'))
    if b is not None:
        out = out + b[None, :, None, None]
    return out


def _residual(x, wa, wb):
    h = jax.nn.relu(_conv2d(x, wa, None, 1, 1))
    return jax.nn.relu(_conv2d(h, wb, None, 1, 0))


_R = 512


def _vq_block_kernel(flat_ref, emb_ref, q_ref, idx_ref, minsum_ref, *,
                     n_valid):
    i = pl.program_id(0)

    @pl.when(i == 0)
    def _init():
        minsum_ref[...] = jnp.zeros_like(minsum_ref)

    f = flat_ref[...]
    e = emb_ref[...]
    rn = jnp.sum(f * f, axis=1, keepdims=True)
    en = jnp.sum(e * e, axis=1)
    mm = jax.lax.dot_general(f, e, (((1,), (1,)), ((), ())),
                             preferred_element_type=jnp.float32)
    d = rn + en[None, :] - 2.0 * mm
    minval = jnp.min(d, axis=1, keepdims=True)
    cidx = jax.lax.broadcasted_iota(jnp.int32, d.shape, 1)
    idx = jnp.min(jnp.where(d == minval, cidx, d.shape[1]), axis=1,
                  keepdims=True)
    idx_ref[...] = idx
    onehot = (cidx == idx).astype(jnp.float32)
    q_ref[...] = jax.lax.dot_general(onehot, e, (((1,), (0,)), ((), ())),
                                     preferred_element_type=jnp.float32)
    rows = i * _R + jax.lax.broadcasted_iota(jnp.int32, (_R, 1), 0)
    vmask = (rows < n_valid).astype(jnp.float32)
    minsum_ref[...] += jnp.sum(minval * vmask).reshape(1, 1)


@functools.cache
def _sc_histogram(n_chunks_a, chunk_a, n_chunks_b, chunk_b, k2,
                  n_valid_a, n_valid_b):
    info = plsc.get_sparse_core_info()
    lanes = info.num_lanes
    nsub = info.num_subcores
    nw = info.num_cores * nsub
    per_w_a = n_chunks_a * chunk_a
    per_w_b = n_chunks_b * chunk_b
    mesh = plsc.VectorSubcoreMesh(core_axis_name="c", subcore_axis_name="s")

    @functools.partial(
        pl.kernel, mesh=mesh,
        out_type=jax.ShapeDtypeStruct((nw, k2), jnp.float32),
        scratch_types=[
            pltpu.VMEM((n_chunks_a, chunk_a), jnp.int32),
            pltpu.VMEM((n_chunks_a, chunk_a), jnp.float32),
            pltpu.VMEM((n_chunks_b, chunk_b), jnp.int32),
            pltpu.VMEM((n_chunks_b, chunk_b), jnp.float32),
            pltpu.VMEM((k2,), jnp.float32),
            pltpu.VMEM((16,), jnp.float32),
            pltpu.VMEM_SHARED((nsub * k2,), jnp.float32),
        ],
    )
    def hist(idxa_hbm, vala_hbm, idxb_hbm, valb_hbm, zero_hbm, out_hbm,
             idxa_v, vala_v, idxb_v, valb_v, cnt_v, acc_v, cnt_sh):
        cid = jax.lax.axis_index("c")
        sid = jax.lax.axis_index("s")
        wid = sid * info.num_cores + cid
        pltpu.sync_copy(idxa_hbm.at[wid], idxa_v)
        pltpu.sync_copy(vala_hbm.at[wid], vala_v)
        pltpu.sync_copy(idxb_hbm.at[wid], idxb_v)
        pltpu.sync_copy(valb_hbm.at[wid], valb_v)
        off = sid * k2
        for c in range(n_chunks_a):
            for j in range(chunk_a // lanes):
                idxa_v[c, pl.ds(j * lanes, lanes)] = (
                    idxa_v[c, pl.ds(j * lanes, lanes)] + off)
        for c in range(n_chunks_b):
            for j in range(chunk_b // lanes):
                idxb_v[c, pl.ds(j * lanes, lanes)] = (
                    idxb_v[c, pl.ds(j * lanes, lanes)] + off)
        pltpu.sync_copy(zero_hbm, cnt_sh.at[pl.ds(off, k2)])
        for c in range(n_chunks_a):
            pltpu.sync_copy(vala_v.at[c], cnt_sh.at[idxa_v.at[c]], add=True)
        for c in range(n_chunks_b):
            pltpu.sync_copy(valb_v.at[c], cnt_sh.at[idxb_v.at[c]], add=True)
        pl.delay(1 << 18)
        pltpu.sync_copy(cnt_sh.at[pl.ds(off, k2)], cnt_v)
        pltpu.sync_copy(cnt_v, out_hbm.at[wid])

    return hist


_NW = 32


def _hist_slices(idx2, n, offset):
    n_p = idx2.shape[0]
    per_w = n_p // _NW
    if per_w <= 128:
        chunk = per_w
    else:
        chunk = next(c for c in range(96, 7, -8) if per_w % c == 0)
    n_chunks = per_w // chunk
    idx3 = idx2.reshape(_NW, n_chunks, chunk) + offset
    val3 = (jnp.arange(n_p, dtype=jnp.int32) < n).astype(jnp.float32).reshape(
        _NW, n_chunks, chunk)
    return idx3, val3, n_chunks, chunk


def _vq_quantize(flat, emb):
    n, dim = flat.shape
    k = emb.shape[0]
    npad = (-n) % _R
    flat_p = jnp.pad(flat, ((0, npad), (0, 0)))
    n_p = n + npad
    q, idx2, minsum = pl.pallas_call(
        functools.partial(_vq_block_kernel, n_valid=n),
        grid=(n_p // _R,),
        in_specs=[pl.BlockSpec((_R, dim), lambda i: (i, 0)),
                  pl.BlockSpec((k, dim), lambda i: (0, 0))],
        out_specs=[pl.BlockSpec((_R, dim), lambda i: (i, 0)),
                   pl.BlockSpec((_R, 1), lambda i: (i, 0)),
                   pl.BlockSpec((1, 1), lambda i: (0, 0))],
        out_shape=[jax.ShapeDtypeStruct((n_p, dim), jnp.float32),
                   jax.ShapeDtypeStruct((n_p, 1), jnp.int32),
                   jax.ShapeDtypeStruct((1, 1), jnp.float32)],
    )(flat_p, emb)
    return q[:n], minsum[0, 0], idx2


def _vq(z, emb):
    x = jnp.transpose(z, (0, 2, 3, 1))
    shp = x.shape
    flat = x.reshape(-1, emb.shape[1])
    n = flat.shape[0]
    q, minsum, idx2 = _vq_quantize(flat, emb)
    loss = 1.25 * (minsum / (n * emb.shape[1]))
    qr = q.reshape(shp)
    q_st = x + (qr - x)
    return loss, jnp.transpose(q_st, (0, 3, 1, 2)), idx2, n


def _perplexities(idx_top, n_top, idx_bot, n_bot, k):
    ia, va, nca, ca = _hist_slices(idx_top, n_top, 0)
    ib, vb, ncb, cb = _hist_slices(idx_bot, n_bot, k)
    hist = _sc_histogram(nca, ca, ncb, cb, 2 * k, n_top, n_bot)(
        ia, va, ib, vb, jnp.zeros((2 * k,), jnp.float32))
    counts = jnp.sum(hist, axis=0)

    def perp(c, n):
        avg = c / n
        return jnp.exp(-jnp.sum(avg * jnp.log(avg + 1e-10)))

    return perp(counts[:k], n_top), perp(counts[k:], n_bot)


def kernel(x, params):
    p = params
    h = jax.nn.relu(_conv2d(x, p['eb_c1_w'], p['eb_c1_b'], 2, 1))
    h = jax.nn.relu(_conv2d(h, p['eb_c2_w'], p['eb_c2_b'], 2, 1))
    h = jax.nn.relu(_conv2d(h, p['eb_c3_w'], p['eb_c3_b'], 1, 1))
    h = _residual(h, p['eb_r1a_w'], p['eb_r1b_w'])
    z_bottom = _residual(h, p['eb_r2a_w'], p['eb_r2b_w'])
    h = jax.nn.relu(_conv2d(z_bottom, p['et_c1_w'], p['et_c1_b'], 2, 1))
    h = jax.nn.relu(_conv2d(h, p['et_c2_w'], p['et_c2_b'], 1, 1))
    h = _residual(h, p['et_r1a_w'], p['et_r1b_w'])
    z_top = _residual(h, p['et_r2a_w'], p['et_r2b_w'])
    loss_top, q_top, idx_top, n_top = _vq(
        _conv2d(z_top, p['pvt_w'], p['pvt_b'], 1, 0), p['emb_top'])
    h = _conv2d(q_top, p['dt_c1_w'], p['dt_c1_b'], 1, 1)
    h = _residual(h, p['dt_r1a_w'], p['dt_r1b_w'])
    h = _residual(h, p['dt_r2a_w'], p['dt_r2b_w'])
    rec_top = _conv_transpose2d(h, p['dt_t1_w'], p['dt_t1_b'], 2, 1)
    zb = jnp.concatenate([rec_top, z_bottom], axis=1)
    loss_bottom, q_bot, idx_bot, n_bot = _vq(
        _conv2d(zb, p['pvb_w'], p['pvb_b'], 1, 0), p['emb_bot'])
    pt, pb = _perplexities(idx_top, n_top, idx_bot, n_bot,
                           p['emb_top'].shape[0])
    up = _conv_transpose2d(q_top, p['up_w'], p['up_b'], 2, 1)
    quantized = jnp.concatenate([up, q_bot], axis=1)
    h = _conv2d(quantized, p['db_c1_w'], p['db_c1_b'], 1, 1)
    h = _residual(h, p['db_r1a_w'], p['db_r1b_w'])
    h = _residual(h, p['db_r2a_w'], p['db_r2b_w'])
    h = jax.nn.relu(_conv_transpose2d(h, p['db_t1_w'], p['db_t1_b'], 2, 1))
    x_rec = _conv_transpose2d(h, p['db_t2_w'], p['db_t2_b'], 2, 1)
    return loss_top + loss_bottom, x_rec, pt + pb, quantized


# speedup vs baseline: 1.6967x; 1.0840x over previous
import functools

import jax
import jax.numpy as jnp
from jax.experimental import pallas as pl
from jax.experimental.pallas import tpu as pltpu
from jax.experimental.pallas import tpu_sc as plsc


def _conv2d(x, w, b, stride, pad):
    out = jax.lax.conv_general_dilated(
        x, w, (stride, stride), [(pad, pad), (pad, pad)],
        dimension_numbers=('NCHW', 'OIHW', 'NCHW'))
    if b is not None:
        out = out + b[None, :, None, None]
    return out


def _conv_transpose2d(x, w, b, stride, pad):
    w_t = jnp.transpose(w[:, :, ::-1, ::-1], (1, 0, 2, 3))
    k = w.shape[2]
    p = k - 1 - pad
    out = jax.lax.conv_general_dilated(
        x, w_t, (1, 1), [(p, p), (p, p)], lhs_dilation=(stride, stride),
        dimension_numbers=('NCHW', 'OIHW', 'NCHW'))
    if b is not None:
        out = out + b[None, :, None, None]
    return out


def _residual(x, wa, wb):
    h = jax.nn.relu(_conv2d(x, wa, None, 1, 1))
    return jax.nn.relu(_conv2d(h, wb, None, 1, 0))


_R = 512


def _vq_block_kernel(flat_ref, emb_ref, q_ref, idx_ref, minsum_ref, *,
                     n_valid):
    i = pl.program_id(0)

    @pl.when(i == 0)
    def _init():
        minsum_ref[...] = jnp.zeros_like(minsum_ref)

    f = flat_ref[...]
    e = emb_ref[...]
    rn = jnp.sum(f * f, axis=1, keepdims=True)
    en = jnp.sum(e * e, axis=1)
    mm = jax.lax.dot_general(f, e, (((1,), (1,)), ((), ())),
                             preferred_element_type=jnp.float32)
    d = rn + en[None, :] - 2.0 * mm
    minval = jnp.min(d, axis=1, keepdims=True)
    cidx = jax.lax.broadcasted_iota(jnp.int32, d.shape, 1)
    idx = jnp.min(jnp.where(d == minval, cidx, d.shape[1]), axis=1,
                  keepdims=True)
    idx_ref[...] = idx
    onehot = (cidx == idx).astype(jnp.float32)
    q_ref[...] = jax.lax.dot_general(onehot, e, (((1,), (0,)), ((), ())),
                                     preferred_element_type=jnp.float32)
    rows = i * _R + jax.lax.broadcasted_iota(jnp.int32, (_R, 1), 0)
    vmask = (rows < n_valid).astype(jnp.float32)
    minsum_ref[...] += jnp.sum(minval * vmask).reshape(1, 1)


@functools.cache
def _sc_histogram(n_chunks_a, chunk_a, n_chunks_b, chunk_b, k2,
                  n_valid_a, n_valid_b):
    info = plsc.get_sparse_core_info()
    lanes = info.num_lanes
    nsub = info.num_subcores
    nw = info.num_cores * nsub
    per_w_a = n_chunks_a * chunk_a
    per_w_b = n_chunks_b * chunk_b
    mesh = plsc.VectorSubcoreMesh(core_axis_name="c", subcore_axis_name="s")

    @functools.partial(
        pl.kernel, mesh=mesh,
        out_type=jax.ShapeDtypeStruct((nw, k2), jnp.float32),
        scratch_types=[
            pltpu.VMEM((n_chunks_a, chunk_a), jnp.int32),
            pltpu.VMEM((n_chunks_a, chunk_a), jnp.float32),
            pltpu.VMEM((n_chunks_b, chunk_b), jnp.int32),
            pltpu.VMEM((n_chunks_b, chunk_b), jnp.float32),
            pltpu.VMEM((k2,), jnp.float32),
            pltpu.VMEM((16,), jnp.float32),
            pltpu.VMEM_SHARED((nsub * k2,), jnp.float32),
        ],
    )
    def hist(idxa_hbm, vala_hbm, idxb_hbm, valb_hbm, zero_hbm, out_hbm,
             idxa_v, vala_v, idxb_v, valb_v, cnt_v, acc_v, cnt_sh):
        cid = jax.lax.axis_index("c")
        sid = jax.lax.axis_index("s")
        wid = sid * info.num_cores + cid
        pltpu.sync_copy(idxa_hbm.at[wid], idxa_v)
        pltpu.sync_copy(vala_hbm.at[wid], vala_v)
        pltpu.sync_copy(idxb_hbm.at[wid], idxb_v)
        pltpu.sync_copy(valb_hbm.at[wid], valb_v)
        off = sid * k2
        for c in range(n_chunks_a):
            for j in range(chunk_a // lanes):
                idxa_v[c, pl.ds(j * lanes, lanes)] = (
                    idxa_v[c, pl.ds(j * lanes, lanes)] + off)
        for c in range(n_chunks_b):
            for j in range(chunk_b // lanes):
                idxb_v[c, pl.ds(j * lanes, lanes)] = (
                    idxb_v[c, pl.ds(j * lanes, lanes)] + off)
        pltpu.sync_copy(zero_hbm, cnt_sh.at[pl.ds(off, k2)])
        for c in range(n_chunks_a):
            pltpu.sync_copy(vala_v.at[c], cnt_sh.at[idxa_v.at[c]], add=True)
        for c in range(n_chunks_b):
            pltpu.sync_copy(valb_v.at[c], cnt_sh.at[idxb_v.at[c]], add=True)
        pl.delay(1 << 17)
        pltpu.sync_copy(cnt_sh.at[pl.ds(off, k2)], cnt_v)
        pltpu.sync_copy(cnt_v, out_hbm.at[wid])

    return hist


_NW = 32


def _hist_slices(idx2, n, offset):
    n_p = idx2.shape[0]
    per_w = n_p // _NW
    if per_w <= 128:
        chunk = per_w
    else:
        chunk = next(c for c in range(96, 7, -8) if per_w % c == 0)
    n_chunks = per_w // chunk
    idx3 = idx2.reshape(_NW, n_chunks, chunk) + offset
    val3 = (jnp.arange(n_p, dtype=jnp.int32) < n).astype(jnp.float32).reshape(
        _NW, n_chunks, chunk)
    return idx3, val3, n_chunks, chunk


def _vq_quantize(flat, emb):
    n, dim = flat.shape
    k = emb.shape[0]
    npad = (-n) % _R
    flat_p = jnp.pad(flat, ((0, npad), (0, 0)))
    n_p = n + npad
    q, idx2, minsum = pl.pallas_call(
        functools.partial(_vq_block_kernel, n_valid=n),
        grid=(n_p // _R,),
        in_specs=[pl.BlockSpec((_R, dim), lambda i: (i, 0)),
                  pl.BlockSpec((k, dim), lambda i: (0, 0))],
        out_specs=[pl.BlockSpec((_R, dim), lambda i: (i, 0)),
                   pl.BlockSpec((_R, 1), lambda i: (i, 0)),
                   pl.BlockSpec((1, 1), lambda i: (0, 0))],
        out_shape=[jax.ShapeDtypeStruct((n_p, dim), jnp.float32),
                   jax.ShapeDtypeStruct((n_p, 1), jnp.int32),
                   jax.ShapeDtypeStruct((1, 1), jnp.float32)],
    )(flat_p, emb)
    return q[:n], minsum[0, 0], idx2


def _vq(z, emb):
    x = jnp.transpose(z, (0, 2, 3, 1))
    shp = x.shape
    flat = x.reshape(-1, emb.shape[1])
    n = flat.shape[0]
    q, minsum, idx2 = _vq_quantize(flat, emb)
    loss = 1.25 * (minsum / (n * emb.shape[1]))
    qr = q.reshape(shp)
    q_st = x + (qr - x)
    return loss, jnp.transpose(q_st, (0, 3, 1, 2)), idx2, n


def _perplexities(idx_top, n_top, idx_bot, n_bot, k):
    ia, va, nca, ca = _hist_slices(idx_top, n_top, 0)
    ib, vb, ncb, cb = _hist_slices(idx_bot, n_bot, k)
    hist = _sc_histogram(nca, ca, ncb, cb, 2 * k, n_top, n_bot)(
        ia, va, ib, vb, jnp.zeros((2 * k,), jnp.float32))
    counts = jnp.sum(hist, axis=0)

    def perp(c, n):
        avg = c / n
        return jnp.exp(-jnp.sum(avg * jnp.log(avg + 1e-10)))

    return perp(counts[:k], n_top), perp(counts[k:], n_bot)


def kernel(x, params):
    p = params
    h = jax.nn.relu(_conv2d(x, p['eb_c1_w'], p['eb_c1_b'], 2, 1))
    h = jax.nn.relu(_conv2d(h, p['eb_c2_w'], p['eb_c2_b'], 2, 1))
    h = jax.nn.relu(_conv2d(h, p['eb_c3_w'], p['eb_c3_b'], 1, 1))
    h = _residual(h, p['eb_r1a_w'], p['eb_r1b_w'])
    z_bottom = _residual(h, p['eb_r2a_w'], p['eb_r2b_w'])
    h = jax.nn.relu(_conv2d(z_bottom, p['et_c1_w'], p['et_c1_b'], 2, 1))
    h = jax.nn.relu(_conv2d(h, p['et_c2_w'], p['et_c2_b'], 1, 1))
    h = _residual(h, p['et_r1a_w'], p['et_r1b_w'])
    z_top = _residual(h, p['et_r2a_w'], p['et_r2b_w'])
    loss_top, q_top, idx_top, n_top = _vq(
        _conv2d(z_top, p['pvt_w'], p['pvt_b'], 1, 0), p['emb_top'])
    h = _conv2d(q_top, p['dt_c1_w'], p['dt_c1_b'], 1, 1)
    h = _residual(h, p['dt_r1a_w'], p['dt_r1b_w'])
    h = _residual(h, p['dt_r2a_w'], p['dt_r2b_w'])
    rec_top = _conv_transpose2d(h, p['dt_t1_w'], p['dt_t1_b'], 2, 1)
    zb = jnp.concatenate([rec_top, z_bottom], axis=1)
    loss_bottom, q_bot, idx_bot, n_bot = _vq(
        _conv2d(zb, p['pvb_w'], p['pvb_b'], 1, 0), p['emb_bot'])
    pt, pb = _perplexities(idx_top, n_top, idx_bot, n_bot,
                           p['emb_top'].shape[0])
    up = _conv_transpose2d(q_top, p['up_w'], p['up_b'], 2, 1)
    quantized = jnp.concatenate([up, q_bot], axis=1)
    h = _conv2d(quantized, p['db_c1_w'], p['db_c1_b'], 1, 1)
    h = _residual(h, p['db_r1a_w'], p['db_r1b_w'])
    h = _residual(h, p['db_r2a_w'], p['db_r2b_w'])
    h = jax.nn.relu(_conv_transpose2d(h, p['db_t1_w'], p['db_t1_b'], 2, 1))
    x_rec = _conv_transpose2d(h, p['db_t2_w'], p['db_t2_b'], 2, 1)
    return loss_top + loss_bottom, x_rec, pt + pb, quantized


# speedup vs baseline: 1.6975x; 1.0005x over previous
import functools

import jax
import jax.numpy as jnp
from jax.experimental import pallas as pl
from jax.experimental.pallas import tpu as pltpu
from jax.experimental.pallas import tpu_sc as plsc


def _conv2d(x, w, b, stride, pad):
    out = jax.lax.conv_general_dilated(
        x, w, (stride, stride), [(pad, pad), (pad, pad)],
        dimension_numbers=('NCHW', 'OIHW', 'NCHW'))
    if b is not None:
        out = out + b[None, :, None, None]
    return out


def _conv_transpose2d(x, w, b, stride, pad):
    w_t = jnp.transpose(w[:, :, ::-1, ::-1], (1, 0, 2, 3))
    k = w.shape[2]
    p = k - 1 - pad
    out = jax.lax.conv_general_dilated(
        x, w_t, (1, 1), [(p, p), (p, p)], lhs_dilation=(stride, stride),
        dimension_numbers=('NCHW', 'OIHW', 'NCHW'))
    if b is not None:
        out = out + b[None, :, None, None]
    return out


def _residual(x, wa, wb):
    h = jax.nn.relu(_conv2d(x, wa, None, 1, 1))
    return jax.nn.relu(_conv2d(h, wb, None, 1, 0))


_R = 512


def _vq_block_kernel(flat_ref, emb_ref, q_ref, idx_ref, minsum_ref, *,
                     n_valid):
    i = pl.program_id(0)

    @pl.when(i == 0)
    def _init():
        minsum_ref[...] = jnp.zeros_like(minsum_ref)

    f = flat_ref[...]
    e = emb_ref[...]
    rn = jnp.sum(f * f, axis=1, keepdims=True)
    en = jnp.sum(e * e, axis=1)
    mm = jax.lax.dot_general(f, e, (((1,), (1,)), ((), ())),
                             preferred_element_type=jnp.float32)
    d = rn + en[None, :] - 2.0 * mm
    minval = jnp.min(d, axis=1, keepdims=True)
    cidx = jax.lax.broadcasted_iota(jnp.int32, d.shape, 1)
    idx = jnp.min(jnp.where(d == minval, cidx, d.shape[1]), axis=1,
                  keepdims=True)
    idx_ref[...] = idx
    onehot = (cidx == idx).astype(jnp.float32)
    q_ref[...] = jax.lax.dot_general(onehot, e, (((1,), (0,)), ((), ())),
                                     preferred_element_type=jnp.float32)
    rows = i * _R + jax.lax.broadcasted_iota(jnp.int32, (_R, 1), 0)
    vmask = (rows < n_valid).astype(jnp.float32)
    minsum_ref[...] += jnp.sum(minval * vmask).reshape(1, 1)


@functools.cache
def _sc_histogram(n_chunks_a, chunk_a, n_chunks_b, chunk_b, k2,
                  n_valid_a, n_valid_b):
    info = plsc.get_sparse_core_info()
    lanes = info.num_lanes
    nsub = info.num_subcores
    nw = info.num_cores * nsub
    per_w_a = n_chunks_a * chunk_a
    per_w_b = n_chunks_b * chunk_b
    mesh = plsc.VectorSubcoreMesh(core_axis_name="c", subcore_axis_name="s")

    @functools.partial(
        pl.kernel, mesh=mesh,
        out_type=jax.ShapeDtypeStruct((nw, k2), jnp.float32),
        scratch_types=[
            pltpu.VMEM((n_chunks_a, chunk_a), jnp.int32),
            pltpu.VMEM((n_chunks_a, chunk_a), jnp.float32),
            pltpu.VMEM((n_chunks_b, chunk_b), jnp.int32),
            pltpu.VMEM((n_chunks_b, chunk_b), jnp.float32),
            pltpu.VMEM((k2,), jnp.float32),
            pltpu.VMEM((16,), jnp.float32),
            pltpu.VMEM_SHARED((nsub * k2,), jnp.float32),
        ],
    )
    def hist(idxa_hbm, vala_hbm, idxb_hbm, valb_hbm, zero_hbm, out_hbm,
             idxa_v, vala_v, idxb_v, valb_v, cnt_v, acc_v, cnt_sh):
        cid = jax.lax.axis_index("c")
        sid = jax.lax.axis_index("s")
        wid = sid * info.num_cores + cid
        pltpu.sync_copy(idxa_hbm.at[wid], idxa_v)
        pltpu.sync_copy(vala_hbm.at[wid], vala_v)
        pltpu.sync_copy(idxb_hbm.at[wid], idxb_v)
        pltpu.sync_copy(valb_hbm.at[wid], valb_v)
        off = sid * k2
        for c in range(n_chunks_a):
            for j in range(chunk_a // lanes):
                idxa_v[c, pl.ds(j * lanes, lanes)] = (
                    idxa_v[c, pl.ds(j * lanes, lanes)] + off)
        for c in range(n_chunks_b):
            for j in range(chunk_b // lanes):
                idxb_v[c, pl.ds(j * lanes, lanes)] = (
                    idxb_v[c, pl.ds(j * lanes, lanes)] + off)
        pltpu.sync_copy(zero_hbm, cnt_sh.at[pl.ds(off, k2)])
        for c in range(n_chunks_a):
            pltpu.sync_copy(vala_v.at[c], cnt_sh.at[idxa_v.at[c]], add=True)
        for c in range(n_chunks_b):
            pltpu.sync_copy(valb_v.at[c], cnt_sh.at[idxb_v.at[c]], add=True)
        pl.delay(1 << 16)
        pltpu.sync_copy(cnt_sh.at[pl.ds(off, k2)], cnt_v)
        pltpu.sync_copy(cnt_v, out_hbm.at[wid])

    return hist


_NW = 32


def _hist_slices(idx2, n, offset):
    n_p = idx2.shape[0]
    per_w = n_p // _NW
    if per_w <= 128:
        chunk = per_w
    else:
        chunk = next(c for c in range(96, 7, -8) if per_w % c == 0)
    n_chunks = per_w // chunk
    idx3 = idx2.reshape(_NW, n_chunks, chunk) + offset
    val3 = (jnp.arange(n_p, dtype=jnp.int32) < n).astype(jnp.float32).reshape(
        _NW, n_chunks, chunk)
    return idx3, val3, n_chunks, chunk


def _vq_quantize(flat, emb):
    n, dim = flat.shape
    k = emb.shape[0]
    npad = (-n) % _R
    flat_p = jnp.pad(flat, ((0, npad), (0, 0)))
    n_p = n + npad
    q, idx2, minsum = pl.pallas_call(
        functools.partial(_vq_block_kernel, n_valid=n),
        grid=(n_p // _R,),
        in_specs=[pl.BlockSpec((_R, dim), lambda i: (i, 0)),
                  pl.BlockSpec((k, dim), lambda i: (0, 0))],
        out_specs=[pl.BlockSpec((_R, dim), lambda i: (i, 0)),
                   pl.BlockSpec((_R, 1), lambda i: (i, 0)),
                   pl.BlockSpec((1, 1), lambda i: (0, 0))],
        out_shape=[jax.ShapeDtypeStruct((n_p, dim), jnp.float32),
                   jax.ShapeDtypeStruct((n_p, 1), jnp.int32),
                   jax.ShapeDtypeStruct((1, 1), jnp.float32)],
    )(flat_p, emb)
    return q[:n], minsum[0, 0], idx2


def _vq(z, emb):
    x = jnp.transpose(z, (0, 2, 3, 1))
    shp = x.shape
    flat = x.reshape(-1, emb.shape[1])
    n = flat.shape[0]
    q, minsum, idx2 = _vq_quantize(flat, emb)
    loss = 1.25 * (minsum / (n * emb.shape[1]))
    qr = q.reshape(shp)
    q_st = x + (qr - x)
    return loss, jnp.transpose(q_st, (0, 3, 1, 2)), idx2, n


def _perplexities(idx_top, n_top, idx_bot, n_bot, k):
    ia, va, nca, ca = _hist_slices(idx_top, n_top, 0)
    ib, vb, ncb, cb = _hist_slices(idx_bot, n_bot, k)
    hist = _sc_histogram(nca, ca, ncb, cb, 2 * k, n_top, n_bot)(
        ia, va, ib, vb, jnp.zeros((2 * k,), jnp.float32))
    counts = jnp.sum(hist, axis=0)

    def perp(c, n):
        avg = c / n
        return jnp.exp(-jnp.sum(avg * jnp.log(avg + 1e-10)))

    return perp(counts[:k], n_top), perp(counts[k:], n_bot)


def kernel(x, params):
    p = params
    h = jax.nn.relu(_conv2d(x, p['eb_c1_w'], p['eb_c1_b'], 2, 1))
    h = jax.nn.relu(_conv2d(h, p['eb_c2_w'], p['eb_c2_b'], 2, 1))
    h = jax.nn.relu(_conv2d(h, p['eb_c3_w'], p['eb_c3_b'], 1, 1))
    h = _residual(h, p['eb_r1a_w'], p['eb_r1b_w'])
    z_bottom = _residual(h, p['eb_r2a_w'], p['eb_r2b_w'])
    h = jax.nn.relu(_conv2d(z_bottom, p['et_c1_w'], p['et_c1_b'], 2, 1))
    h = jax.nn.relu(_conv2d(h, p['et_c2_w'], p['et_c2_b'], 1, 1))
    h = _residual(h, p['et_r1a_w'], p['et_r1b_w'])
    z_top = _residual(h, p['et_r2a_w'], p['et_r2b_w'])
    loss_top, q_top, idx_top, n_top = _vq(
        _conv2d(z_top, p['pvt_w'], p['pvt_b'], 1, 0), p['emb_top'])
    h = _conv2d(q_top, p['dt_c1_w'], p['dt_c1_b'], 1, 1)
    h = _residual(h, p['dt_r1a_w'], p['dt_r1b_w'])
    h = _residual(h, p['dt_r2a_w'], p['dt_r2b_w'])
    rec_top = _conv_transpose2d(h, p['dt_t1_w'], p['dt_t1_b'], 2, 1)
    zb = jnp.concatenate([rec_top, z_bottom], axis=1)
    loss_bottom, q_bot, idx_bot, n_bot = _vq(
        _conv2d(zb, p['pvb_w'], p['pvb_b'], 1, 0), p['emb_bot'])
    pt, pb = _perplexities(idx_top, n_top, idx_bot, n_bot,
                           p['emb_top'].shape[0])
    up = _conv_transpose2d(q_top, p['up_w'], p['up_b'], 2, 1)
    quantized = jnp.concatenate([up, q_bot], axis=1)
    h = _conv2d(quantized, p['db_c1_w'], p['db_c1_b'], 1, 1)
    h = _residual(h, p['db_r1a_w'], p['db_r1b_w'])
    h = _residual(h, p['db_r2a_w'], p['db_r2b_w'])
    h = jax.nn.relu(_conv_transpose2d(h, p['db_t1_w'], p['db_t1_b'], 2, 1))
    x_rec = _conv_transpose2d(h, p['db_t2_w'], p['db_t2_b'], 2, 1)
    return loss_top + loss_bottom, x_rec, pt + pb, quantized
